# baseline (speedup 1.0000x reference)
import functools

import jax
import jax.numpy as jnp
from jax import lax
from jax.experimental import pallas as pl
from jax.experimental.pallas import tpu as pltpu
from jax.experimental.pallas import tpu_sc as plsc

T = 2048
D = 768
H = 3072
E = 8
K = 2
B = 256
NB = T * K // B + (E - 1)
NSLOT = NB * B

NW = 32
TW = T // NW
CH = 512


def _route_body(x_ref, wr_ref, br_ref,
                pos0_ref, pos1_ref, w0_ref, w1_ref, be_ref, nbu_ref,
                chg_ref, par_ref, nxt_ref, hn_ref):
    x = x_ref[...]
    logits = jnp.dot(x, wr_ref[...], preferred_element_type=jnp.float32)
    logits = logits + br_ref[...]
    m = jnp.max(logits, axis=1, keepdims=True)
    ex = jnp.exp(logits - m)
    p = ex / jnp.sum(ex, axis=1, keepdims=True)

    eids = lax.broadcasted_iota(jnp.int32, (T, E), 1)
    p1 = jnp.max(p, axis=1, keepdims=True)
    i1 = jnp.min(jnp.where(p == p1, eids, E), axis=1, keepdims=True)
    pm = jnp.where(eids == i1, -jnp.inf, p)
    p2 = jnp.max(pm, axis=1, keepdims=True)
    i2 = jnp.min(jnp.where(pm == p2, eids, E), axis=1, keepdims=True)

    oh1t = jnp.transpose((eids == i1).astype(jnp.float32))
    oh2t = jnp.transpose((eids == i2).astype(jnp.float32))
    oht = oh1t + oh2t

    rr = lax.broadcasted_iota(jnp.int32, (CH, CH), 0)
    cc = lax.broadcasted_iota(jnp.int32, (CH, CH), 1)
    su_tri = (rr < cc).astype(jnp.float32)
    carry = jnp.zeros((E, 1), jnp.float32)
    pieces = []
    for c in range(T // CH):
        chunk = lax.slice(oht, (0, c * CH), (E, (c + 1) * CH))
        pieces.append(
            jnp.dot(chunk, su_tri, preferred_element_type=jnp.float32) + carry)
        carry = carry + jnp.sum(chunk, axis=1, keepdims=True)
    cnt_before = jnp.concatenate(pieces, axis=1)
    cnt = carry

    padded = jnp.floor((cnt + (B - 1)) / B) * B
    er = lax.broadcasted_iota(jnp.int32, (E, E), 0)
    ec = lax.broadcasted_iota(jnp.int32, (E, E), 1)
    sl_tri = (ec < er).astype(jnp.float32)
    off = jnp.dot(sl_tri, padded, preferred_element_type=jnp.float32)
    total = jnp.sum(padded, axis=0, keepdims=True)

    tgt = off + cnt_before
    pos0_ref[...] = jnp.sum(oh1t * tgt, axis=0, keepdims=True
                            ).astype(jnp.int32).reshape(T)
    pos1_ref[...] = jnp.sum(oh2t * tgt, axis=0, keepdims=True
                            ).astype(jnp.int32).reshape(T)
    w0_ref[...] = jnp.broadcast_to(p1, (T, 128))
    w1_ref[...] = jnp.broadcast_to(p2, (T, 128))

    jb = lax.broadcasted_iota(jnp.int32, (1, NB), 1).astype(jnp.float32) * B
    jbc = jnp.minimum(jb, total - 1.0)
    acc = jnp.zeros((1, NB), jnp.int32)
    for e in range(E):
        off_e = lax.slice(off, (e, 0), (e + 1, 1))
        acc = acc + (off_e <= jbc).astype(jnp.int32)
    be = acc - 1
    be_ref[...] = be
    nbu_ref[...] = (total / B).astype(jnp.int32)

    jidx = lax.broadcasted_iota(jnp.int32, (1, NB), 1)
    be_prev = jnp.concatenate(
        [lax.slice(be, (0, 0), (1, 1)), lax.slice(be, (0, 0), (1, NB - 1))],
        axis=1)
    used = (jb < total)
    chg = (((be != be_prev) | (jidx == 0)) & used).astype(jnp.int32)
    nr = lax.broadcasted_iota(jnp.int32, (NB, NB), 0)
    nc = lax.broadcasted_iota(jnp.int32, (NB, NB), 1)
    su_nb = (nr < nc).astype(jnp.float32)
    cum = jnp.dot(chg.astype(jnp.float32), su_nb,
                  preferred_element_type=jnp.float32) + chg.astype(jnp.float32)
    par = (cum - 2.0 * jnp.floor(cum * 0.5)).astype(jnp.int32)
    par = 1 - par
    paddedT = jnp.transpose(padded)
    nxte = jnp.min(jnp.where((ec > er) & (paddedT > 0.0), ec, E),
                   axis=1, keepdims=True)
    nxt = jnp.zeros((1, NB), jnp.int32)
    hn = jnp.zeros((1, NB), jnp.int32)
    for e in range(E):
        sel = (be == e).astype(jnp.int32)
        ne_e = lax.slice(nxte, (e, 0), (e + 1, 1)).astype(jnp.int32)
        nxt = nxt + sel * jnp.minimum(ne_e, E - 1)
        hn = hn + sel * (ne_e < E).astype(jnp.int32)
    chg_ref[...] = chg
    par_ref[...] = par
    nxt_ref[...] = nxt
    hn_ref[...] = hn


_route = pl.pallas_call(
    _route_body,
    out_shape=(
        jax.ShapeDtypeStruct((T,), jnp.int32),
        jax.ShapeDtypeStruct((T,), jnp.int32),
        jax.ShapeDtypeStruct((T, 128), jnp.float32),
        jax.ShapeDtypeStruct((T, 128), jnp.float32),
        jax.ShapeDtypeStruct((1, NB), jnp.int32),
        jax.ShapeDtypeStruct((1, 1), jnp.int32),
        jax.ShapeDtypeStruct((1, NB), jnp.int32),
        jax.ShapeDtypeStruct((1, NB), jnp.int32),
        jax.ShapeDtypeStruct((1, NB), jnp.int32),
        jax.ShapeDtypeStruct((1, NB), jnp.int32),
    ),
)


@functools.cache
def _make_dispatch():
    mesh = plsc.VectorSubcoreMesh(core_axis_name="c", subcore_axis_name="s")

    @functools.partial(
        pl.kernel,
        out_type=(
            jax.ShapeDtypeStruct((NSLOT, D), jnp.float32),
            jax.ShapeDtypeStruct((NSLOT, 128), jnp.float32),
        ),
        mesh=mesh,
        scratch_types=[
            pltpu.VMEM((TW, D), jnp.float32),
            pltpu.VMEM((TW, 128), jnp.float32),
            pltpu.VMEM((TW, 128), jnp.float32),
            pltpu.VMEM((TW,), jnp.int32),
            pltpu.VMEM((TW,), jnp.int32),
            pltpu.SemaphoreType.DMA,
        ],
    )
    def _dispatch(x_hbm, pos0_hbm, pos1_hbm, w0_hbm, w1_hbm, xs_hbm, ws_hbm,
                  xv, wv0, wv1, i0v, i1v, sem):
        wid = lax.axis_index("s") * 2 + lax.axis_index("c")
        base = wid * TW
        pltpu.sync_copy(x_hbm.at[pl.ds(base, TW), :], xv)
        pltpu.sync_copy(pos0_hbm.at[pl.ds(base, TW)], i0v)
        pltpu.sync_copy(pos1_hbm.at[pl.ds(base, TW)], i1v)
        pltpu.sync_copy(w0_hbm.at[pl.ds(base, TW), :], wv0)
        pltpu.sync_copy(w1_hbm.at[pl.ds(base, TW), :], wv1)
        cp0 = pltpu.async_copy(xv, xs_hbm.at[i0v], sem)
        cp1 = pltpu.async_copy(xv, xs_hbm.at[i1v], sem)
        cp2 = pltpu.async_copy(wv0, ws_hbm.at[i0v], sem)
        cp3 = pltpu.async_copy(wv1, ws_hbm.at[i1v], sem)
        cp0.wait()
        cp1.wait()
        cp2.wait()
        cp3.wait()

    return _dispatch


def _ffn_body(be_ref, nbu_ref, chg_ref, par_ref, nxt_ref, hn_ref,
              xs_ref, ws_ref, w1_hbm, b1_ref, w2_hbm, b2_ref, ys_ref,
              w1s, w2s, sem1, sem2):
    j = pl.program_id(0)
    nbu = nbu_ref[0]
    e = be_ref[j]
    par = par_ref[j]

    def w_copies(src_e, dst):
        c1 = pltpu.make_async_copy(w1_hbm.at[src_e], w1s.at[dst], sem1)
        c2 = pltpu.make_async_copy(w2_hbm.at[src_e], w2s.at[dst], sem2)
        return c1, c2

    @pl.when(j == 0)
    def _():
        c1, c2 = w_copies(e, 0)
        c1.start()
        c2.start()

    @pl.when(chg_ref[j] == 1)
    def _():
        c1, c2 = w_copies(e, par)
        c1.wait()
        c2.wait()

        @pl.when(hn_ref[j] == 1)
        def _():
            p1c, p2c = w_copies(nxt_ref[j], 1 - par)
            p1c.start()
            p2c.start()

    @pl.when(j < nbu)
    def _():
        xb = xs_ref[...]
        h = jnp.dot(xb, w1s[par], preferred_element_type=jnp.float32)
        h = h + b1_ref[0]
        h = 0.5 * h * (1.0 + lax.erf(h * 0.7071067811865476))
        y = jnp.dot(h, w2s[par], preferred_element_type=jnp.float32)
        ys_ref[...] = (y + b2_ref[0]) * ws_ref[...][:, 0:1]


def _clamp_j(j, *sc):
    return jnp.minimum(j, sc[1][0] - 1)


_ffn = pl.pallas_call(
    _ffn_body,
    grid_spec=pltpu.PrefetchScalarGridSpec(
        num_scalar_prefetch=6,
        grid=(NB,),
        in_specs=[
            pl.BlockSpec((B, D), lambda j, *sc: (_clamp_j(j, *sc), 0)),
            pl.BlockSpec((B, 128), lambda j, *sc: (_clamp_j(j, *sc), 0)),
            pl.BlockSpec(memory_space=pl.ANY),
            pl.BlockSpec((1, 1, H), lambda j, *sc: (sc[0][j], 0, 0)),
            pl.BlockSpec(memory_space=pl.ANY),
            pl.BlockSpec((1, 1, D), lambda j, *sc: (sc[0][j], 0, 0)),
        ],
        out_specs=pl.BlockSpec((B, D), lambda j, *sc: (_clamp_j(j, *sc), 0)),
        scratch_shapes=[
            pltpu.VMEM((2, D, H), jnp.float32),
            pltpu.VMEM((2, H, D), jnp.float32),
            pltpu.SemaphoreType.DMA,
            pltpu.SemaphoreType.DMA,
        ],
    ),
    out_shape=jax.ShapeDtypeStruct((NSLOT, D), jnp.float32),
)


@functools.cache
def _make_combine():
    mesh = plsc.VectorSubcoreMesh(core_axis_name="c", subcore_axis_name="s")

    @functools.partial(
        pl.kernel,
        out_type=jax.ShapeDtypeStruct((T, D), jnp.float32),
        mesh=mesh,
        scratch_types=[
            pltpu.VMEM((TW, D), jnp.float32),
            pltpu.VMEM((TW, D), jnp.float32),
            pltpu.VMEM((TW,), jnp.int32),
            pltpu.VMEM((TW,), jnp.int32),
            pltpu.SemaphoreType.DMA,
        ],
    )
    def _combine(ys_hbm, pos0_hbm, pos1_hbm, out_hbm, av, bv, i0v, i1v, sem):
        wid = lax.axis_index("s") * 2 + lax.axis_index("c")
        base = wid * TW
        pltpu.sync_copy(pos0_hbm.at[pl.ds(base, TW)], i0v)
        pltpu.sync_copy(pos1_hbm.at[pl.ds(base, TW)], i1v)
        cpa = pltpu.async_copy(ys_hbm.at[i0v], av, sem)
        cpb = pltpu.async_copy(ys_hbm.at[i1v], bv, sem)
        cpa.wait()
        cpb.wait()

        @plsc.parallel_loop(0, TW, unroll=2)
        def _(jj):
            for c in range(D // 16):
                sl = pl.ds(c * 16, 16)
                av[jj, sl] = av[jj, sl] + bv[jj, sl]

        pltpu.sync_copy(av, out_hbm.at[pl.ds(base, TW), :])

    return _combine


def kernel(x, Wr, br, W1, b1, W2, b2):
    (pos0, pos1, w0, w1, be, nbu,
     chg, par, nxt, hn) = _route(x, Wr, br.reshape(1, E))
    xs, ws = _make_dispatch()(x, pos0, pos1, w0, w1)
    ys = _ffn(be.reshape(NB), nbu.reshape(1), chg.reshape(NB),
              par.reshape(NB), nxt.reshape(NB), hn.reshape(NB),
              xs, ws, W1, b1.reshape(E, 1, H), W2, b2.reshape(E, 1, D))
    return _make_combine()(ys, pos0, pos1)


# speedup vs baseline: 1.0075x; 1.0075x over previous
import functools

import jax
import jax.numpy as jnp
from jax import lax
from jax.experimental import pallas as pl
from jax.experimental.pallas import tpu as pltpu
from jax.experimental.pallas import tpu_sc as plsc

T = 2048
D = 768
H = 3072
E = 8
K = 2
B = 256
NB = T * K // B + (E - 1)
NSLOT = NB * B

NW = 32
TW = T // NW
CH = 512


def _route_body(x_ref, wr_ref, br_ref,
                pos0_ref, pos1_ref, w0_ref, w1_ref, be_ref, nbu_ref,
                chg_ref, par_ref, nxt_ref, hn_ref):
    x = x_ref[...]
    logits = jnp.dot(x, wr_ref[...], preferred_element_type=jnp.float32)
    logits = logits + br_ref[...]
    m = jnp.max(logits, axis=1, keepdims=True)
    ex = jnp.exp(logits - m)
    p = ex / jnp.sum(ex, axis=1, keepdims=True)

    eids = lax.broadcasted_iota(jnp.int32, (T, E), 1)
    p1 = jnp.max(p, axis=1, keepdims=True)
    i1 = jnp.min(jnp.where(p == p1, eids, E), axis=1, keepdims=True)
    pm = jnp.where(eids == i1, -jnp.inf, p)
    p2 = jnp.max(pm, axis=1, keepdims=True)
    i2 = jnp.min(jnp.where(pm == p2, eids, E), axis=1, keepdims=True)

    oh1t = jnp.transpose((eids == i1).astype(jnp.float32))
    oh2t = jnp.transpose((eids == i2).astype(jnp.float32))
    oht = oh1t + oh2t

    rr = lax.broadcasted_iota(jnp.int32, (CH, CH), 0)
    cc = lax.broadcasted_iota(jnp.int32, (CH, CH), 1)
    su_tri = (rr < cc).astype(jnp.float32)
    carry = jnp.zeros((E, 1), jnp.float32)
    pieces = []
    for c in range(T // CH):
        chunk = lax.slice(oht, (0, c * CH), (E, (c + 1) * CH))
        pieces.append(
            jnp.dot(chunk, su_tri, preferred_element_type=jnp.float32) + carry)
        carry = carry + jnp.sum(chunk, axis=1, keepdims=True)
    cnt_before = jnp.concatenate(pieces, axis=1)
    cnt = carry

    padded = jnp.floor((cnt + (B - 1)) / B) * B
    er = lax.broadcasted_iota(jnp.int32, (E, E), 0)
    ec = lax.broadcasted_iota(jnp.int32, (E, E), 1)
    sl_tri = (ec < er).astype(jnp.float32)
    off = jnp.dot(sl_tri, padded, preferred_element_type=jnp.float32)
    total = jnp.sum(padded, axis=0, keepdims=True)

    tgt = off + cnt_before
    pos0_ref[...] = jnp.sum(oh1t * tgt, axis=0, keepdims=True
                            ).astype(jnp.int32).reshape(T)
    pos1_ref[...] = jnp.sum(oh2t * tgt, axis=0, keepdims=True
                            ).astype(jnp.int32).reshape(T)
    w0_ref[...] = jnp.broadcast_to(p1, (T, 128))
    w1_ref[...] = jnp.broadcast_to(p2, (T, 128))

    jb = lax.broadcasted_iota(jnp.int32, (1, NB), 1).astype(jnp.float32) * B
    jbc = jnp.minimum(jb, total - 1.0)
    acc = jnp.zeros((1, NB), jnp.int32)
    for e in range(E):
        off_e = lax.slice(off, (e, 0), (e + 1, 1))
        acc = acc + (off_e <= jbc).astype(jnp.int32)
    be = acc - 1
    be_ref[...] = be
    nbu_ref[...] = (total / B).astype(jnp.int32)

    jidx = lax.broadcasted_iota(jnp.int32, (1, NB), 1)
    be_prev = jnp.concatenate(
        [lax.slice(be, (0, 0), (1, 1)), lax.slice(be, (0, 0), (1, NB - 1))],
        axis=1)
    used = (jb < total)
    chg = (((be != be_prev) | (jidx == 0)) & used).astype(jnp.int32)
    nr = lax.broadcasted_iota(jnp.int32, (NB, NB), 0)
    nc = lax.broadcasted_iota(jnp.int32, (NB, NB), 1)
    su_nb = (nr < nc).astype(jnp.float32)
    cum = jnp.dot(chg.astype(jnp.float32), su_nb,
                  preferred_element_type=jnp.float32) + chg.astype(jnp.float32)
    par = (cum - 2.0 * jnp.floor(cum * 0.5)).astype(jnp.int32)
    par = 1 - par
    paddedT = jnp.transpose(padded)
    nxte = jnp.min(jnp.where((ec > er) & (paddedT > 0.0), ec, E),
                   axis=1, keepdims=True)
    nxt = jnp.zeros((1, NB), jnp.int32)
    hn = jnp.zeros((1, NB), jnp.int32)
    for e in range(E):
        sel = (be == e).astype(jnp.int32)
        ne_e = lax.slice(nxte, (e, 0), (e + 1, 1)).astype(jnp.int32)
        nxt = nxt + sel * jnp.minimum(ne_e, E - 1)
        hn = hn + sel * (ne_e < E).astype(jnp.int32)
    chg_ref[...] = chg
    par_ref[...] = par
    nxt_ref[...] = nxt
    hn_ref[...] = hn


_route = pl.pallas_call(
    _route_body,
    out_shape=(
        jax.ShapeDtypeStruct((T,), jnp.int32),
        jax.ShapeDtypeStruct((T,), jnp.int32),
        jax.ShapeDtypeStruct((T, 128), jnp.float32),
        jax.ShapeDtypeStruct((T, 128), jnp.float32),
        jax.ShapeDtypeStruct((1, NB), jnp.int32),
        jax.ShapeDtypeStruct((1, 1), jnp.int32),
        jax.ShapeDtypeStruct((1, NB), jnp.int32),
        jax.ShapeDtypeStruct((1, NB), jnp.int32),
        jax.ShapeDtypeStruct((1, NB), jnp.int32),
        jax.ShapeDtypeStruct((1, NB), jnp.int32),
    ),
)


@functools.cache
def _make_dispatch():
    mesh = plsc.VectorSubcoreMesh(core_axis_name="c", subcore_axis_name="s")

    @functools.partial(
        pl.kernel,
        out_type=(
            jax.ShapeDtypeStruct((NSLOT, D), jnp.float32),
            jax.ShapeDtypeStruct((NSLOT, 128), jnp.float32),
        ),
        mesh=mesh,
        scratch_types=[
            pltpu.VMEM((TW, D), jnp.float32),
            pltpu.VMEM((TW, 128), jnp.float32),
            pltpu.VMEM((TW, 128), jnp.float32),
            pltpu.VMEM((TW,), jnp.int32),
            pltpu.VMEM((TW,), jnp.int32),
            pltpu.SemaphoreType.DMA,
        ],
    )
    def _dispatch(x_hbm, pos0_hbm, pos1_hbm, w0_hbm, w1_hbm, xs_hbm, ws_hbm,
                  xv, wv0, wv1, i0v, i1v, sem):
        wid = lax.axis_index("s") * 2 + lax.axis_index("c")
        base = wid * TW
        pltpu.sync_copy(x_hbm.at[pl.ds(base, TW), :], xv)
        pltpu.sync_copy(pos0_hbm.at[pl.ds(base, TW)], i0v)
        pltpu.sync_copy(pos1_hbm.at[pl.ds(base, TW)], i1v)
        pltpu.sync_copy(w0_hbm.at[pl.ds(base, TW), :], wv0)
        pltpu.sync_copy(w1_hbm.at[pl.ds(base, TW), :], wv1)
        cp0 = pltpu.async_copy(xv, xs_hbm.at[i0v], sem)
        cp1 = pltpu.async_copy(xv, xs_hbm.at[i1v], sem)
        cp2 = pltpu.async_copy(wv0, ws_hbm.at[i0v], sem)
        cp3 = pltpu.async_copy(wv1, ws_hbm.at[i1v], sem)
        cp0.wait()
        cp1.wait()
        cp2.wait()
        cp3.wait()

    return _dispatch


def _ffn_body(be_ref, nbu_ref, chg_ref, par_ref, nxt_ref, hn_ref,
              xs_ref, ws_ref, w1_hbm, b1_ref, w2_hbm, b2_ref, ys_ref,
              w1s, w2s, sem1, sem2):
    j = pl.program_id(0)
    nbu = nbu_ref[0]
    e = be_ref[j]
    par = par_ref[j]

    def w_copies(src_e, dst):
        c1 = pltpu.make_async_copy(w1_hbm.at[src_e], w1s.at[dst], sem1)
        c2 = pltpu.make_async_copy(w2_hbm.at[src_e], w2s.at[dst], sem2)
        return c1, c2

    @pl.when(j == 0)
    def _():
        c1, c2 = w_copies(e, 0)
        c1.start()
        c2.start()

    @pl.when(chg_ref[j] == 1)
    def _():
        c1, c2 = w_copies(e, par)
        c1.wait()
        c2.wait()

        @pl.when(hn_ref[j] == 1)
        def _():
            p1c, p2c = w_copies(nxt_ref[j], 1 - par)
            p1c.start()
            p2c.start()

    @pl.when(j < nbu)
    def _():
        xb = xs_ref[...]
        h = jnp.dot(xb, w1s[par], preferred_element_type=jnp.float32)
        h = h + b1_ref[0]
        h = 0.5 * h * (1.0 + lax.erf(h * 0.7071067811865476))
        y = jnp.dot(h, w2s[par], preferred_element_type=jnp.float32)
        ys_ref[...] = (y + b2_ref[0]) * ws_ref[...][:, 0:1]


def _clamp_j(j, *sc):
    return jnp.minimum(j, sc[1][0] - 1)


_ffn = pl.pallas_call(
    _ffn_body,
    grid_spec=pltpu.PrefetchScalarGridSpec(
        num_scalar_prefetch=6,
        grid=(NB,),
        in_specs=[
            pl.BlockSpec((B, D), lambda j, *sc: (_clamp_j(j, *sc), 0)),
            pl.BlockSpec((B, 128), lambda j, *sc: (_clamp_j(j, *sc), 0)),
            pl.BlockSpec(memory_space=pl.ANY),
            pl.BlockSpec((1, 1, H), lambda j, *sc: (sc[0][j], 0, 0)),
            pl.BlockSpec(memory_space=pl.ANY),
            pl.BlockSpec((1, 1, D), lambda j, *sc: (sc[0][j], 0, 0)),
        ],
        out_specs=pl.BlockSpec((B, D), lambda j, *sc: (_clamp_j(j, *sc), 0)),
        scratch_shapes=[
            pltpu.VMEM((2, D, H), jnp.float32),
            pltpu.VMEM((2, H, D), jnp.float32),
            pltpu.SemaphoreType.DMA,
            pltpu.SemaphoreType.DMA,
        ],
    ),
    out_shape=jax.ShapeDtypeStruct((NSLOT, D), jnp.float32),
)


@functools.cache
def _make_combine():
    mesh = plsc.VectorSubcoreMesh(core_axis_name="c", subcore_axis_name="s")

    @functools.partial(
        pl.kernel,
        out_type=jax.ShapeDtypeStruct((T, D), jnp.float32),
        mesh=mesh,
        scratch_types=[
            pltpu.VMEM((TW, D), jnp.float32),
            pltpu.VMEM((TW, D), jnp.float32),
            pltpu.VMEM((TW,), jnp.int32),
            pltpu.VMEM((TW,), jnp.int32),
            pltpu.SemaphoreType.DMA,
        ],
    )
    def _combine(ys_hbm, pos0_hbm, pos1_hbm, out_hbm, av, bv, i0v, i1v, sem):
        wid = lax.axis_index("s") * 2 + lax.axis_index("c")
        base = wid * TW
        pltpu.sync_copy(pos0_hbm.at[pl.ds(base, TW)], i0v)
        pltpu.sync_copy(pos1_hbm.at[pl.ds(base, TW)], i1v)
        cpa = pltpu.async_copy(ys_hbm.at[i0v], av, sem)
        cpb = pltpu.async_copy(ys_hbm.at[i1v], bv, sem)
        cpa.wait()
        cpb.wait()

        def row_body(jj, _):
            for c in range(D // 16):
                sl = pl.ds(c * 16, 16)
                av[jj, sl] = av[jj, sl] + bv[jj, sl]
            return 0

        lax.fori_loop(0, TW, row_body, 0)
        pltpu.sync_copy(av, out_hbm.at[pl.ds(base, TW), :])

    return _combine


def kernel(x, Wr, br, W1, b1, W2, b2):
    (pos0, pos1, w0, w1, be, nbu,
     chg, par, nxt, hn) = _route(x, Wr, br.reshape(1, E))
    xs, ws = _make_dispatch()(x, pos0, pos1, w0, w1)
    ys = _ffn(be.reshape(NB), nbu.reshape(1), chg.reshape(NB),
              par.reshape(NB), nxt.reshape(NB), hn.reshape(NB),
              xs, ws, W1, b1.reshape(E, 1, H), W2, b2.reshape(E, 1, D))
    return _make_combine()(ys, pos0, pos1)


# speedup vs baseline: 1.0817x; 1.0737x over previous
import functools

import jax
import jax.numpy as jnp
from jax import lax
from jax.experimental import pallas as pl
from jax.experimental.pallas import tpu as pltpu
from jax.experimental.pallas import tpu_sc as plsc

T = 2048
D = 768
H = 3072
E = 8
K = 2
B = 320
NB = -(-T * K // B) + (E - 1)
NSLOT = NB * B

NW = 32
TW = T // NW
CH = 512


def _route_body(x_ref, wr_ref, br_ref,
                pos0_ref, pos1_ref, w0_ref, w1_ref, be_ref, nbu_ref,
                chg_ref, par_ref, nxt_ref, hn_ref):
    x = x_ref[...]
    logits = jnp.dot(x, wr_ref[...], preferred_element_type=jnp.float32)
    logits = logits + br_ref[...]
    m = jnp.max(logits, axis=1, keepdims=True)
    ex = jnp.exp(logits - m)
    p = ex / jnp.sum(ex, axis=1, keepdims=True)

    eids = lax.broadcasted_iota(jnp.int32, (T, E), 1)
    p1 = jnp.max(p, axis=1, keepdims=True)
    i1 = jnp.min(jnp.where(p == p1, eids, E), axis=1, keepdims=True)
    pm = jnp.where(eids == i1, -jnp.inf, p)
    p2 = jnp.max(pm, axis=1, keepdims=True)
    i2 = jnp.min(jnp.where(pm == p2, eids, E), axis=1, keepdims=True)

    oh1t = jnp.transpose((eids == i1).astype(jnp.float32))
    oh2t = jnp.transpose((eids == i2).astype(jnp.float32))
    oht = oh1t + oh2t

    rr = lax.broadcasted_iota(jnp.int32, (CH, CH), 0)
    cc = lax.broadcasted_iota(jnp.int32, (CH, CH), 1)
    su_tri = (rr < cc).astype(jnp.float32)
    carry = jnp.zeros((E, 1), jnp.float32)
    pieces = []
    for c in range(T // CH):
        chunk = lax.slice(oht, (0, c * CH), (E, (c + 1) * CH))
        pieces.append(
            jnp.dot(chunk, su_tri, preferred_element_type=jnp.float32) + carry)
        carry = carry + jnp.sum(chunk, axis=1, keepdims=True)
    cnt_before = jnp.concatenate(pieces, axis=1)
    cnt = carry

    padded = jnp.floor((cnt + (B - 1)) / B) * B
    er = lax.broadcasted_iota(jnp.int32, (E, E), 0)
    ec = lax.broadcasted_iota(jnp.int32, (E, E), 1)
    sl_tri = (ec < er).astype(jnp.float32)
    off = jnp.dot(sl_tri, padded, preferred_element_type=jnp.float32)
    total = jnp.sum(padded, axis=0, keepdims=True)

    tgt = off + cnt_before
    pos0_ref[...] = jnp.sum(oh1t * tgt, axis=0, keepdims=True
                            ).astype(jnp.int32).reshape(T)
    pos1_ref[...] = jnp.sum(oh2t * tgt, axis=0, keepdims=True
                            ).astype(jnp.int32).reshape(T)
    w0_ref[...] = jnp.broadcast_to(p1, (T, 128))
    w1_ref[...] = jnp.broadcast_to(p2, (T, 128))

    jb = lax.broadcasted_iota(jnp.int32, (1, NB), 1).astype(jnp.float32) * B
    jbc = jnp.minimum(jb, total - 1.0)
    acc = jnp.zeros((1, NB), jnp.int32)
    for e in range(E):
        off_e = lax.slice(off, (e, 0), (e + 1, 1))
        acc = acc + (off_e <= jbc).astype(jnp.int32)
    be = acc - 1
    be_ref[...] = be
    nbu_ref[...] = (total / B).astype(jnp.int32)

    jidx = lax.broadcasted_iota(jnp.int32, (1, NB), 1)
    be_prev = jnp.concatenate(
        [lax.slice(be, (0, 0), (1, 1)), lax.slice(be, (0, 0), (1, NB - 1))],
        axis=1)
    used = (jb < total)
    chg = (((be != be_prev) | (jidx == 0)) & used).astype(jnp.int32)
    nr = lax.broadcasted_iota(jnp.int32, (NB, NB), 0)
    nc = lax.broadcasted_iota(jnp.int32, (NB, NB), 1)
    su_nb = (nr < nc).astype(jnp.float32)
    cum = jnp.dot(chg.astype(jnp.float32), su_nb,
                  preferred_element_type=jnp.float32) + chg.astype(jnp.float32)
    par = (cum - 2.0 * jnp.floor(cum * 0.5)).astype(jnp.int32)
    par = 1 - par
    paddedT = jnp.transpose(padded)
    nxte = jnp.min(jnp.where((ec > er) & (paddedT > 0.0), ec, E),
                   axis=1, keepdims=True)
    nxt = jnp.zeros((1, NB), jnp.int32)
    hn = jnp.zeros((1, NB), jnp.int32)
    for e in range(E):
        sel = (be == e).astype(jnp.int32)
        ne_e = lax.slice(nxte, (e, 0), (e + 1, 1)).astype(jnp.int32)
        nxt = nxt + sel * jnp.minimum(ne_e, E - 1)
        hn = hn + sel * (ne_e < E).astype(jnp.int32)
    chg_ref[...] = chg
    par_ref[...] = par
    nxt_ref[...] = nxt
    hn_ref[...] = hn


_route = pl.pallas_call(
    _route_body,
    out_shape=(
        jax.ShapeDtypeStruct((T,), jnp.int32),
        jax.ShapeDtypeStruct((T,), jnp.int32),
        jax.ShapeDtypeStruct((T, 128), jnp.float32),
        jax.ShapeDtypeStruct((T, 128), jnp.float32),
        jax.ShapeDtypeStruct((1, NB), jnp.int32),
        jax.ShapeDtypeStruct((1, 1), jnp.int32),
        jax.ShapeDtypeStruct((1, NB), jnp.int32),
        jax.ShapeDtypeStruct((1, NB), jnp.int32),
        jax.ShapeDtypeStruct((1, NB), jnp.int32),
        jax.ShapeDtypeStruct((1, NB), jnp.int32),
    ),
)


@functools.cache
def _make_dispatch():
    mesh = plsc.VectorSubcoreMesh(core_axis_name="c", subcore_axis_name="s")

    @functools.partial(
        pl.kernel,
        out_type=(
            jax.ShapeDtypeStruct((NSLOT, D), jnp.float32),
            jax.ShapeDtypeStruct((NSLOT, 128), jnp.float32),
        ),
        mesh=mesh,
        scratch_types=[
            pltpu.VMEM((TW, D), jnp.float32),
            pltpu.VMEM((TW, 128), jnp.float32),
            pltpu.VMEM((TW, 128), jnp.float32),
            pltpu.VMEM((TW,), jnp.int32),
            pltpu.VMEM((TW,), jnp.int32),
            pltpu.SemaphoreType.DMA,
        ],
    )
    def _dispatch(x_hbm, pos0_hbm, pos1_hbm, w0_hbm, w1_hbm, xs_hbm, ws_hbm,
                  xv, wv0, wv1, i0v, i1v, sem):
        wid = lax.axis_index("s") * 2 + lax.axis_index("c")
        base = wid * TW
        pltpu.sync_copy(x_hbm.at[pl.ds(base, TW), :], xv)
        pltpu.sync_copy(pos0_hbm.at[pl.ds(base, TW)], i0v)
        pltpu.sync_copy(pos1_hbm.at[pl.ds(base, TW)], i1v)
        pltpu.sync_copy(w0_hbm.at[pl.ds(base, TW), :], wv0)
        pltpu.sync_copy(w1_hbm.at[pl.ds(base, TW), :], wv1)
        cp0 = pltpu.async_copy(xv, xs_hbm.at[i0v], sem)
        cp1 = pltpu.async_copy(xv, xs_hbm.at[i1v], sem)
        cp2 = pltpu.async_copy(wv0, ws_hbm.at[i0v], sem)
        cp3 = pltpu.async_copy(wv1, ws_hbm.at[i1v], sem)
        cp0.wait()
        cp1.wait()
        cp2.wait()
        cp3.wait()

    return _dispatch


def _ffn_body(be_ref, nbu_ref, chg_ref, par_ref, nxt_ref, hn_ref,
              xs_ref, ws_ref, w1_hbm, b1_ref, w2_hbm, b2_ref, ys_ref,
              w1s, w2s, sem1, sem2):
    j = pl.program_id(0)
    nbu = nbu_ref[0]
    e = be_ref[j]
    par = par_ref[j]

    def w_copies(src_e, dst):
        c1 = pltpu.make_async_copy(w1_hbm.at[src_e], w1s.at[dst], sem1)
        c2 = pltpu.make_async_copy(w2_hbm.at[src_e], w2s.at[dst], sem2)
        return c1, c2

    @pl.when(j == 0)
    def _():
        c1, c2 = w_copies(e, 0)
        c1.start()
        c2.start()

    @pl.when(chg_ref[j] == 1)
    def _():
        c1, c2 = w_copies(e, par)
        c1.wait()
        c2.wait()

        @pl.when(hn_ref[j] == 1)
        def _():
            p1c, p2c = w_copies(nxt_ref[j], 1 - par)
            p1c.start()
            p2c.start()

    @pl.when(j < nbu)
    def _():
        xb = xs_ref[...]
        h = jnp.dot(xb, w1s[par], preferred_element_type=jnp.float32)
        h = h + b1_ref[0]
        h = 0.5 * h * (1.0 + lax.erf(h * 0.7071067811865476))
        y = jnp.dot(h, w2s[par], preferred_element_type=jnp.float32)
        ys_ref[...] = (y + b2_ref[0]) * ws_ref[...][:, 0:1]


def _clamp_j(j, *sc):
    return jnp.minimum(j, sc[1][0] - 1)


_ffn = pl.pallas_call(
    _ffn_body,
    grid_spec=pltpu.PrefetchScalarGridSpec(
        num_scalar_prefetch=6,
        grid=(NB,),
        in_specs=[
            pl.BlockSpec((B, D), lambda j, *sc: (_clamp_j(j, *sc), 0)),
            pl.BlockSpec((B, 128), lambda j, *sc: (_clamp_j(j, *sc), 0)),
            pl.BlockSpec(memory_space=pl.ANY),
            pl.BlockSpec((1, 1, H), lambda j, *sc: (sc[0][j], 0, 0)),
            pl.BlockSpec(memory_space=pl.ANY),
            pl.BlockSpec((1, 1, D), lambda j, *sc: (sc[0][j], 0, 0)),
        ],
        out_specs=pl.BlockSpec((B, D), lambda j, *sc: (_clamp_j(j, *sc), 0)),
        scratch_shapes=[
            pltpu.VMEM((2, D, H), jnp.float32),
            pltpu.VMEM((2, H, D), jnp.float32),
            pltpu.SemaphoreType.DMA,
            pltpu.SemaphoreType.DMA,
        ],
    ),
    out_shape=jax.ShapeDtypeStruct((NSLOT, D), jnp.float32),
)


@functools.cache
def _make_combine():
    mesh = plsc.VectorSubcoreMesh(core_axis_name="c", subcore_axis_name="s")

    @functools.partial(
        pl.kernel,
        out_type=jax.ShapeDtypeStruct((T, D), jnp.float32),
        mesh=mesh,
        scratch_types=[
            pltpu.VMEM((TW, D), jnp.float32),
            pltpu.VMEM((TW, D), jnp.float32),
            pltpu.VMEM((TW,), jnp.int32),
            pltpu.VMEM((TW,), jnp.int32),
            pltpu.SemaphoreType.DMA,
        ],
    )
    def _combine(ys_hbm, pos0_hbm, pos1_hbm, out_hbm, av, bv, i0v, i1v, sem):
        wid = lax.axis_index("s") * 2 + lax.axis_index("c")
        base = wid * TW
        pltpu.sync_copy(pos0_hbm.at[pl.ds(base, TW)], i0v)
        pltpu.sync_copy(pos1_hbm.at[pl.ds(base, TW)], i1v)
        cpa = pltpu.async_copy(ys_hbm.at[i0v], av, sem)
        cpb = pltpu.async_copy(ys_hbm.at[i1v], bv, sem)
        cpa.wait()
        cpb.wait()

        def row_body(jj, _):
            for c in range(D // 16):
                sl = pl.ds(c * 16, 16)
                av[jj, sl] = av[jj, sl] + bv[jj, sl]
            return 0

        lax.fori_loop(0, TW, row_body, 0)
        pltpu.sync_copy(av, out_hbm.at[pl.ds(base, TW), :])

    return _combine


def kernel(x, Wr, br, W1, b1, W2, b2):
    (pos0, pos1, w0, w1, be, nbu,
     chg, par, nxt, hn) = _route(x, Wr, br.reshape(1, E))
    xs, ws = _make_dispatch()(x, pos0, pos1, w0, w1)
    ys = _ffn(be.reshape(NB), nbu.reshape(1), chg.reshape(NB),
              par.reshape(NB), nxt.reshape(NB), hn.reshape(NB),
              xs, ws, W1, b1.reshape(E, 1, H), W2, b2.reshape(E, 1, D))
    return _make_combine()(ys, pos0, pos1)


# speedup vs baseline: 1.1070x; 1.0234x over previous
import functools

import jax
import jax.numpy as jnp
from jax import lax
from jax.experimental import pallas as pl
from jax.experimental.pallas import tpu as pltpu
from jax.experimental.pallas import tpu_sc as plsc

T = 2048
D = 768
H = 3072
E = 8
K = 2
B = 288
NB = -(-T * K // B) + (E - 1)
NSLOT = NB * B

NW = 32
TW = T // NW
CH = 512


def _route_body(x_ref, wr_ref, br_ref,
                pos0_ref, pos1_ref, w0_ref, w1_ref, be_ref, nbu_ref,
                chg_ref, par_ref, nxt_ref, hn_ref):
    x = x_ref[...]
    logits = jnp.dot(x, wr_ref[...], preferred_element_type=jnp.float32)
    logits = logits + br_ref[...]
    m = jnp.max(logits, axis=1, keepdims=True)
    ex = jnp.exp(logits - m)
    p = ex / jnp.sum(ex, axis=1, keepdims=True)

    eids = lax.broadcasted_iota(jnp.int32, (T, E), 1)
    p1 = jnp.max(p, axis=1, keepdims=True)
    i1 = jnp.min(jnp.where(p == p1, eids, E), axis=1, keepdims=True)
    pm = jnp.where(eids == i1, -jnp.inf, p)
    p2 = jnp.max(pm, axis=1, keepdims=True)
    i2 = jnp.min(jnp.where(pm == p2, eids, E), axis=1, keepdims=True)

    oh1t = jnp.transpose((eids == i1).astype(jnp.float32))
    oh2t = jnp.transpose((eids == i2).astype(jnp.float32))
    oht = oh1t + oh2t

    rr = lax.broadcasted_iota(jnp.int32, (CH, CH), 0)
    cc = lax.broadcasted_iota(jnp.int32, (CH, CH), 1)
    su_tri = (rr < cc).astype(jnp.float32)
    carry = jnp.zeros((E, 1), jnp.float32)
    pieces = []
    for c in range(T // CH):
        chunk = lax.slice(oht, (0, c * CH), (E, (c + 1) * CH))
        pieces.append(
            jnp.dot(chunk, su_tri, preferred_element_type=jnp.float32) + carry)
        carry = carry + jnp.sum(chunk, axis=1, keepdims=True)
    cnt_before = jnp.concatenate(pieces, axis=1)
    cnt = carry

    padded = jnp.floor((cnt + (B - 1)) / B) * B
    er = lax.broadcasted_iota(jnp.int32, (E, E), 0)
    ec = lax.broadcasted_iota(jnp.int32, (E, E), 1)
    sl_tri = (ec < er).astype(jnp.float32)
    off = jnp.dot(sl_tri, padded, preferred_element_type=jnp.float32)
    total = jnp.sum(padded, axis=0, keepdims=True)

    tgt = off + cnt_before
    pos0_ref[...] = jnp.sum(oh1t * tgt, axis=0, keepdims=True
                            ).astype(jnp.int32).reshape(T)
    pos1_ref[...] = jnp.sum(oh2t * tgt, axis=0, keepdims=True
                            ).astype(jnp.int32).reshape(T)
    w0_ref[...] = jnp.broadcast_to(p1, (T, 128))
    w1_ref[...] = jnp.broadcast_to(p2, (T, 128))

    jb = lax.broadcasted_iota(jnp.int32, (1, NB), 1).astype(jnp.float32) * B
    jbc = jnp.minimum(jb, total - 1.0)
    acc = jnp.zeros((1, NB), jnp.int32)
    for e in range(E):
        off_e = lax.slice(off, (e, 0), (e + 1, 1))
        acc = acc + (off_e <= jbc).astype(jnp.int32)
    be = acc - 1
    be_ref[...] = be
    nbu_ref[...] = (total / B).astype(jnp.int32)

    jidx = lax.broadcasted_iota(jnp.int32, (1, NB), 1)
    be_prev = jnp.concatenate(
        [lax.slice(be, (0, 0), (1, 1)), lax.slice(be, (0, 0), (1, NB - 1))],
        axis=1)
    used = (jb < total)
    chg = (((be != be_prev) | (jidx == 0)) & used).astype(jnp.int32)
    nr = lax.broadcasted_iota(jnp.int32, (NB, NB), 0)
    nc = lax.broadcasted_iota(jnp.int32, (NB, NB), 1)
    su_nb = (nr < nc).astype(jnp.float32)
    cum = jnp.dot(chg.astype(jnp.float32), su_nb,
                  preferred_element_type=jnp.float32) + chg.astype(jnp.float32)
    par = (cum - 2.0 * jnp.floor(cum * 0.5)).astype(jnp.int32)
    par = 1 - par
    paddedT = jnp.transpose(padded)
    nxte = jnp.min(jnp.where((ec > er) & (paddedT > 0.0), ec, E),
                   axis=1, keepdims=True)
    nxt = jnp.zeros((1, NB), jnp.int32)
    hn = jnp.zeros((1, NB), jnp.int32)
    for e in range(E):
        sel = (be == e).astype(jnp.int32)
        ne_e = lax.slice(nxte, (e, 0), (e + 1, 1)).astype(jnp.int32)
        nxt = nxt + sel * jnp.minimum(ne_e, E - 1)
        hn = hn + sel * (ne_e < E).astype(jnp.int32)
    chg_ref[...] = chg
    par_ref[...] = par
    nxt_ref[...] = nxt
    hn_ref[...] = hn


_route = pl.pallas_call(
    _route_body,
    out_shape=(
        jax.ShapeDtypeStruct((T,), jnp.int32),
        jax.ShapeDtypeStruct((T,), jnp.int32),
        jax.ShapeDtypeStruct((T, 128), jnp.float32),
        jax.ShapeDtypeStruct((T, 128), jnp.float32),
        jax.ShapeDtypeStruct((1, NB), jnp.int32),
        jax.ShapeDtypeStruct((1, 1), jnp.int32),
        jax.ShapeDtypeStruct((1, NB), jnp.int32),
        jax.ShapeDtypeStruct((1, NB), jnp.int32),
        jax.ShapeDtypeStruct((1, NB), jnp.int32),
        jax.ShapeDtypeStruct((1, NB), jnp.int32),
    ),
)


@functools.cache
def _make_dispatch():
    mesh = plsc.VectorSubcoreMesh(core_axis_name="c", subcore_axis_name="s")

    @functools.partial(
        pl.kernel,
        out_type=(
            jax.ShapeDtypeStruct((NSLOT, D), jnp.float32),
            jax.ShapeDtypeStruct((NSLOT, 128), jnp.float32),
        ),
        mesh=mesh,
        scratch_types=[
            pltpu.VMEM((TW, D), jnp.float32),
            pltpu.VMEM((TW, 128), jnp.float32),
            pltpu.VMEM((TW, 128), jnp.float32),
            pltpu.VMEM((TW,), jnp.int32),
            pltpu.VMEM((TW,), jnp.int32),
            pltpu.SemaphoreType.DMA,
        ],
    )
    def _dispatch(x_hbm, pos0_hbm, pos1_hbm, w0_hbm, w1_hbm, xs_hbm, ws_hbm,
                  xv, wv0, wv1, i0v, i1v, sem):
        wid = lax.axis_index("s") * 2 + lax.axis_index("c")
        base = wid * TW
        pltpu.sync_copy(x_hbm.at[pl.ds(base, TW), :], xv)
        pltpu.sync_copy(pos0_hbm.at[pl.ds(base, TW)], i0v)
        pltpu.sync_copy(pos1_hbm.at[pl.ds(base, TW)], i1v)
        pltpu.sync_copy(w0_hbm.at[pl.ds(base, TW), :], wv0)
        pltpu.sync_copy(w1_hbm.at[pl.ds(base, TW), :], wv1)
        cp0 = pltpu.async_copy(xv, xs_hbm.at[i0v], sem)
        cp1 = pltpu.async_copy(xv, xs_hbm.at[i1v], sem)
        cp2 = pltpu.async_copy(wv0, ws_hbm.at[i0v], sem)
        cp3 = pltpu.async_copy(wv1, ws_hbm.at[i1v], sem)
        cp0.wait()
        cp1.wait()
        cp2.wait()
        cp3.wait()

    return _dispatch


def _ffn_body(be_ref, nbu_ref, chg_ref, par_ref, nxt_ref, hn_ref,
              xs_ref, ws_ref, w1_hbm, b1_ref, w2_hbm, b2_ref, ys_ref,
              w1s, w2s, sem1, sem2):
    j = pl.program_id(0)
    nbu = nbu_ref[0]
    e = be_ref[j]
    par = par_ref[j]

    def w_copies(src_e, dst):
        c1 = pltpu.make_async_copy(w1_hbm.at[src_e], w1s.at[dst], sem1)
        c2 = pltpu.make_async_copy(w2_hbm.at[src_e], w2s.at[dst], sem2)
        return c1, c2

    @pl.when(j == 0)
    def _():
        c1, c2 = w_copies(e, 0)
        c1.start()
        c2.start()

    @pl.when(chg_ref[j] == 1)
    def _():
        c1, c2 = w_copies(e, par)
        c1.wait()
        c2.wait()

        @pl.when(hn_ref[j] == 1)
        def _():
            p1c, p2c = w_copies(nxt_ref[j], 1 - par)
            p1c.start()
            p2c.start()

    @pl.when(j < nbu)
    def _():
        xb = xs_ref[...]
        h = jnp.dot(xb, w1s[par], preferred_element_type=jnp.float32)
        h = h + b1_ref[0]
        h = 0.5 * h * (1.0 + lax.erf(h * 0.7071067811865476))
        y = jnp.dot(h, w2s[par], preferred_element_type=jnp.float32)
        ys_ref[...] = (y + b2_ref[0]) * ws_ref[...][:, 0:1]


def _clamp_j(j, *sc):
    return jnp.minimum(j, sc[1][0] - 1)


_ffn = pl.pallas_call(
    _ffn_body,
    grid_spec=pltpu.PrefetchScalarGridSpec(
        num_scalar_prefetch=6,
        grid=(NB,),
        in_specs=[
            pl.BlockSpec((B, D), lambda j, *sc: (_clamp_j(j, *sc), 0)),
            pl.BlockSpec((B, 128), lambda j, *sc: (_clamp_j(j, *sc), 0)),
            pl.BlockSpec(memory_space=pl.ANY),
            pl.BlockSpec((1, 1, H), lambda j, *sc: (sc[0][j], 0, 0)),
            pl.BlockSpec(memory_space=pl.ANY),
            pl.BlockSpec((1, 1, D), lambda j, *sc: (sc[0][j], 0, 0)),
        ],
        out_specs=pl.BlockSpec((B, D), lambda j, *sc: (_clamp_j(j, *sc), 0)),
        scratch_shapes=[
            pltpu.VMEM((2, D, H), jnp.float32),
            pltpu.VMEM((2, H, D), jnp.float32),
            pltpu.SemaphoreType.DMA,
            pltpu.SemaphoreType.DMA,
        ],
    ),
    out_shape=jax.ShapeDtypeStruct((NSLOT, D), jnp.float32),
)


@functools.cache
def _make_combine():
    mesh = plsc.VectorSubcoreMesh(core_axis_name="c", subcore_axis_name="s")

    @functools.partial(
        pl.kernel,
        out_type=jax.ShapeDtypeStruct((T, D), jnp.float32),
        mesh=mesh,
        scratch_types=[
            pltpu.VMEM((TW, D), jnp.float32),
            pltpu.VMEM((TW, D), jnp.float32),
            pltpu.VMEM((TW,), jnp.int32),
            pltpu.VMEM((TW,), jnp.int32),
            pltpu.SemaphoreType.DMA,
        ],
    )
    def _combine(ys_hbm, pos0_hbm, pos1_hbm, out_hbm, av, bv, i0v, i1v, sem):
        wid = lax.axis_index("s") * 2 + lax.axis_index("c")
        base = wid * TW
        pltpu.sync_copy(pos0_hbm.at[pl.ds(base, TW)], i0v)
        pltpu.sync_copy(pos1_hbm.at[pl.ds(base, TW)], i1v)
        cpa = pltpu.async_copy(ys_hbm.at[i0v], av, sem)
        cpb = pltpu.async_copy(ys_hbm.at[i1v], bv, sem)
        cpa.wait()
        cpb.wait()

        def row_body(jj, _):
            for c in range(D // 16):
                sl = pl.ds(c * 16, 16)
                av[jj, sl] = av[jj, sl] + bv[jj, sl]
            return 0

        lax.fori_loop(0, TW, row_body, 0)
        pltpu.sync_copy(av, out_hbm.at[pl.ds(base, TW), :])

    return _combine


def kernel(x, Wr, br, W1, b1, W2, b2):
    (pos0, pos1, w0, w1, be, nbu,
     chg, par, nxt, hn) = _route(x, Wr, br.reshape(1, E))
    xs, ws = _make_dispatch()(x, pos0, pos1, w0, w1)
    ys = _ffn(be.reshape(NB), nbu.reshape(1), chg.reshape(NB),
              par.reshape(NB), nxt.reshape(NB), hn.reshape(NB),
              xs, ws, W1, b1.reshape(E, 1, H), W2, b2.reshape(E, 1, D))
    return _make_combine()(ys, pos0, pos1)
